# baseline (speedup 1.0000x reference)
import functools

import jax
import jax.numpy as jnp
from jax import lax
from jax.experimental import pallas as pl
from jax.experimental.pallas import tpu as pltpu
from jax.experimental.pallas import tpu_sc as plsc

N = 10000
E = 320000
D = 128
H = 256
O = 16
G = 128

P = 10240
RPT = P // 32
RPS = P // 16
ECH = 80
MSG_NCH = E // 16 // ECH
DEG_NCH = E // 32 // ECH
BR = 512
HH = H // 2

_MESH = plsc.VectorSubcoreMesh(core_axis_name="c", subcore_axis_name="s")


def _zero_vmem(ref, rows, groups):
    z = jnp.zeros((16,), jnp.float32)

    def body(r, _):
        for g in range(groups):
            ref[r, pl.ds(g * 16, 16)] = z
        return 0

    lax.fori_loop(0, rows, body, 0)


def _deg_body(col32, degp, col_v, ones_v, z1, acc1):
    c = lax.axis_index("c")
    s = lax.axis_index("s")
    tid = c * 16 + s
    one = jnp.ones((16,), jnp.float32)
    for g in range(ECH // 16):
        ones_v[pl.ds(g * 16, 16)] = one
    z = jnp.zeros((16,), jnp.float32)

    def zb(i, _):
        z1[pl.ds(i * 16, 16)] = z
        return 0

    lax.fori_loop(0, RPS // 16, zb, 0)
    pltpu.sync_copy(z1, acc1.at[pl.ds(s * RPS, RPS)])
    pltpu.sync_copy(col32.at[tid], col_v)
    plsc.subcore_barrier()

    def chunk(j, _):
        pltpu.sync_copy(ones_v, acc1.at[col_v.at[j]], add=True)
        return 0

    lax.fori_loop(0, DEG_NCH, chunk, 0)
    plsc.subcore_barrier()
    pltpu.sync_copy(acc1.at[pl.ds(s * RPS, RPS)], degp.at[c, pl.ds(s * RPS, RPS)])


_deg_kernel = functools.partial(
    pl.kernel,
    _deg_body,
    out_type=jax.ShapeDtypeStruct((2, P), jnp.float32),
    mesh=_MESH,
    compiler_params=pltpu.CompilerParams(
        use_tc_tiling_on_sc=False, needs_layout_passes=False),
    scratch_types=[
        pltpu.VMEM((DEG_NCH, ECH), jnp.int32),
        pltpu.VMEM((ECH,), jnp.float32),
        pltpu.VMEM((RPS,), jnp.float32),
        pltpu.VMEM_SHARED((P,), jnp.float32),
    ],
)


MCH = 128
EPT = 20480
EP = EPT * 16
TCH = EPT // MCH
SCH = 32
NSC = TCH // SCH


def _msg_body(ya, yb, row16, col16, sa, sb, acc, row_sv, col_sv, gbuf,
              semg0, semg1, sems0, sems1):
    c = lax.axis_index("c")
    s = lax.axis_index("s")

    def run(src, dst):
        z = jnp.zeros((16,), jnp.float32)

        def zb(r, _):
            for g in range(HH // 16):
                gbuf[0, r, pl.ds(g * 16, 16)] = z
            return 0

        lax.fori_loop(0, MCH, zb, 0)
        for k in range(RPS // MCH):
            pltpu.async_copy(
                gbuf.at[0], acc.at[pl.ds(s * RPS + k * MCH, MCH)], sems1)
        for k in range(RPS // MCH):
            pltpu.make_async_copy(
                gbuf.at[0], acc.at[pl.ds(s * RPS + k * MCH, MCH)], sems1).wait()
        plsc.subcore_barrier()

        def superchunk(i, _):
            pltpu.sync_copy(row16.at[s, pl.ds(i * SCH, SCH)], row_sv)
            pltpu.sync_copy(col16.at[s, pl.ds(i * SCH, SCH)], col_sv)
            pltpu.async_copy(src.at[row_sv.at[0]], gbuf.at[0], semg0)
            pltpu.async_copy(src.at[row_sv.at[1]], gbuf.at[1], semg1)

            def pair(k, _):
                j0 = 2 * k
                j1 = j0 + 1
                pltpu.make_async_copy(
                    src.at[row_sv.at[j0]], gbuf.at[0], semg0).wait()
                pltpu.async_copy(
                    gbuf.at[0], acc.at[col_sv.at[j0]], sems0, add=True)
                pltpu.make_async_copy(
                    src.at[row_sv.at[j1]], gbuf.at[1], semg1).wait()
                pltpu.async_copy(
                    gbuf.at[1], acc.at[col_sv.at[j1]], sems1, add=True)
                pltpu.make_async_copy(
                    gbuf.at[0], acc.at[col_sv.at[j0]], sems0).wait()

                @pl.when(k < SCH // 2 - 1)
                def _():
                    pltpu.async_copy(
                        src.at[row_sv.at[j0 + 2]], gbuf.at[0], semg0)

                pltpu.make_async_copy(
                    gbuf.at[1], acc.at[col_sv.at[j1]], sems1).wait()

                @pl.when(k < SCH // 2 - 1)
                def _():
                    pltpu.async_copy(
                        src.at[row_sv.at[j1 + 2]], gbuf.at[1], semg1)

                return 0

            lax.fori_loop(0, SCH // 2, pair, 0)
            return 0

        lax.fori_loop(0, NSC, superchunk, 0)
        plsc.subcore_barrier()
        pltpu.sync_copy(acc.at[pl.ds(s * RPS, RPS)], dst.at[pl.ds(s * RPS, RPS)])

    @pl.when(c == 0)
    def _():
        run(ya, sa)

    @pl.when(c == 1)
    def _():
        run(yb, sb)


_msg_kernel = functools.partial(
    pl.kernel,
    _msg_body,
    out_type=(
        jax.ShapeDtypeStruct((P, HH), jnp.float32),
        jax.ShapeDtypeStruct((P, HH), jnp.float32),
    ),
    mesh=_MESH,
    compiler_params=pltpu.CompilerParams(
        use_tc_tiling_on_sc=False, needs_layout_passes=False),
    scratch_types=[
        pltpu.VMEM_SHARED((P, HH), jnp.float32),
        pltpu.VMEM((SCH, MCH), jnp.int32),
        pltpu.VMEM((SCH, MCH), jnp.int32),
        pltpu.VMEM((2, MCH, HH), jnp.float32),
        pltpu.SemaphoreType.DMA,
        pltpu.SemaphoreType.DMA,
        pltpu.SemaphoreType.DMA,
        pltpu.SemaphoreType.DMA,
    ],
)


def _rsqrt_newton(x):
    i = plsc.bitcast(x, jnp.int32)
    y = plsc.bitcast(jnp.int32(0x5F3759DF) - (i >> 1), jnp.float32)
    for _ in range(3):
        y = y * (1.5 - 0.5 * x * y * y)
    return y


def _pool_body(sa, sb, ya, yb, deg0, deg1, b2, batch, part_out,
               sbufa, sbufb, ybufa, ybufb, dv0, dv1, batch_v, b2_v, part):
    c = lax.axis_index("c")
    s = lax.axis_index("s")
    tid = c * 16 + s
    base = tid * RPT

    pltpu.sync_copy(deg0.at[pl.ds(base, RPT)], dv0.at[pl.ds(0, RPT)])
    pltpu.sync_copy(deg1.at[pl.ds(base, RPT)], dv1.at[pl.ds(0, RPT)])
    pltpu.sync_copy(batch.at[pl.ds(base, RPT)], batch_v.at[pl.ds(0, RPT)])
    pltpu.sync_copy(b2, b2_v)

    def dinvb(i, _):
        deg = dv0[pl.ds(i * 16, 16)] + dv1[pl.ds(i * 16, 16)] + 1.0
        dv0[pl.ds(i * 16, 16)] = _rsqrt_newton(deg)
        return 0

    lax.fori_loop(0, RPT // 16, dinvb, 0)
    _zero_vmem(part, G + 1, H // 16)
    b2vecs = [b2_v[pl.ds(g * 16, 16)] for g in range(H // 16)]

    for ch in range(RPT // 64):
        r0 = ch * 64
        pltpu.sync_copy(sa.at[pl.ds(base + r0, 64)], sbufa)
        pltpu.sync_copy(sb.at[pl.ds(base + r0, 64)], sbufb)
        pltpu.sync_copy(ya.at[pl.ds(base + r0, 64)], ybufa)
        pltpu.sync_copy(yb.at[pl.ds(base + r0, 64)], ybufb)

        def rowb(r, _):
            b_r = batch_v[pl.ds(r0 + r, 16)][0]
            dv = dv0[pl.ds(r0 + r, 16)][0]
            for g in range(H // 16):
                sl = pl.ds((g % 8) * 16, 16)
                if g < 8:
                    v = sbufa[r, sl] + ybufa[r, sl]
                else:
                    v = sbufb[r, sl] + ybufb[r, sl]
                hv = jnp.maximum(v * dv + b2vecs[g], 0.0)
                osl = pl.ds(g * 16, 16)
                part[b_r, osl] = jnp.maximum(part[b_r, osl], hv)
            return 0

        lax.fori_loop(0, 64, rowb, 0)

    pltpu.sync_copy(part, part_out.at[tid])


_pool_kernel = functools.partial(
    pl.kernel,
    _pool_body,
    out_type=jax.ShapeDtypeStruct((32, G + 1, H), jnp.float32),
    mesh=_MESH,
    compiler_params=pltpu.CompilerParams(
        use_tc_tiling_on_sc=False, needs_layout_passes=False),
    scratch_types=[
        pltpu.VMEM((64, HH), jnp.float32),
        pltpu.VMEM((64, HH), jnp.float32),
        pltpu.VMEM((64, HH), jnp.float32),
        pltpu.VMEM((64, HH), jnp.float32),
        pltpu.VMEM((RPT + 16,), jnp.float32),
        pltpu.VMEM((RPT + 16,), jnp.float32),
        pltpu.VMEM((RPT + 16,), jnp.int32),
        pltpu.VMEM((H,), jnp.float32),
        pltpu.VMEM((G + 1, H), jnp.float32),
    ],
)


def _mm1_body(x_ref, w_ref, degp_ref, ya_ref, yb_ref):
    z = jnp.dot(x_ref[...], w_ref[...], preferred_element_type=jnp.float32)
    deg = degp_ref[0, :] + degp_ref[1, :] + 1.0
    dinv = lax.rsqrt(deg)
    y = z * dinv[:, None]
    ya_ref[...] = y[:, :HH]
    yb_ref[...] = y[:, HH:]


def _mm1(x_p, w1, degp):
    return pl.pallas_call(
        _mm1_body,
        grid=(P // BR,),
        in_specs=[
            pl.BlockSpec((BR, D), lambda i: (i, 0)),
            pl.BlockSpec((D, H), lambda i: (0, 0)),
            pl.BlockSpec((2, BR), lambda i: (0, i)),
        ],
        out_specs=[
            pl.BlockSpec((BR, HH), lambda i: (i, 0)),
            pl.BlockSpec((BR, HH), lambda i: (i, 0)),
        ],
        out_shape=[
            jax.ShapeDtypeStruct((P, HH), jnp.float32),
            jax.ShapeDtypeStruct((P, HH), jnp.float32),
        ],
    )(x_p, w1, degp)


def _mm2_body(sa_ref, sb_ref, ya_ref, yb_ref, degp_ref, b1_ref, w2_ref,
              oa_ref, ob_ref):
    deg = degp_ref[0, :] + degp_ref[1, :] + 1.0
    dinv = lax.rsqrt(deg)[:, None]
    ha = jnp.maximum((sa_ref[...] + ya_ref[...]) * dinv + b1_ref[0, :HH], 0.0)
    hb = jnp.maximum((sb_ref[...] + yb_ref[...]) * dinv + b1_ref[0, HH:], 0.0)
    z = jnp.dot(ha, w2_ref[:HH, :], preferred_element_type=jnp.float32)
    z = z + jnp.dot(hb, w2_ref[HH:, :], preferred_element_type=jnp.float32)
    y = z * dinv
    oa_ref[...] = y[:, :HH]
    ob_ref[...] = y[:, HH:]


def _mm2(s1a, s1b, y1a, y1b, degp, b1, w2):
    return pl.pallas_call(
        _mm2_body,
        grid=(P // BR,),
        in_specs=[
            pl.BlockSpec((BR, HH), lambda i: (i, 0)),
            pl.BlockSpec((BR, HH), lambda i: (i, 0)),
            pl.BlockSpec((BR, HH), lambda i: (i, 0)),
            pl.BlockSpec((BR, HH), lambda i: (i, 0)),
            pl.BlockSpec((2, BR), lambda i: (0, i)),
            pl.BlockSpec((1, H), lambda i: (0, 0)),
            pl.BlockSpec((H, H), lambda i: (0, 0)),
        ],
        out_specs=[
            pl.BlockSpec((BR, HH), lambda i: (i, 0)),
            pl.BlockSpec((BR, HH), lambda i: (i, 0)),
        ],
        out_shape=[
            jax.ShapeDtypeStruct((P, HH), jnp.float32),
            jax.ShapeDtypeStruct((P, HH), jnp.float32),
        ],
    )(s1a, s1b, y1a, y1b, degp, b1, w2)


def _out_body(part_ref, wc_ref, bc_ref, out_ref):
    acc = part_ref[0, :G, :]
    for t in range(1, 32):
        acc = jnp.maximum(acc, part_ref[t, :G, :])
    out_ref[...] = (
        jnp.dot(acc, wc_ref[...], preferred_element_type=jnp.float32)
        + bc_ref[0, :]
    )


def _out_mm(part, wc, bc):
    return pl.pallas_call(
        _out_body,
        out_shape=jax.ShapeDtypeStruct((G, O), jnp.float32),
    )(part, wc, bc.reshape(1, O))


def kernel(feature_matrix, edge_index, batch, W1, b1, W2, b2, Wc, bc):
    row = edge_index[0]
    col = edge_index[1]
    x_p = jnp.pad(feature_matrix, ((0, P - N), (0, 0)))
    batch_p = jnp.concatenate(
        [batch, jnp.full((P - N,), G, dtype=batch.dtype)])
    npad_t = EPT - E // 16
    pad_idx = N + (jnp.arange(16 * npad_t, dtype=row.dtype) % (P - N))
    row16 = jnp.concatenate(
        [row.reshape(16, E // 16), pad_idx.reshape(16, npad_t)], axis=1
    ).reshape(16, TCH, MCH)
    col16 = jnp.concatenate(
        [col.reshape(16, E // 16), pad_idx.reshape(16, npad_t)], axis=1
    ).reshape(16, TCH, MCH)
    col32 = col.reshape(32, DEG_NCH, ECH)

    degp = _deg_kernel()(col32)
    y1a, y1b = _mm1(x_p, W1, degp)
    s1a, s1b = _msg_kernel()(y1a, y1b, row16, col16)
    y2a, y2b = _mm2(s1a, s1b, y1a, y1b, degp, b1.reshape(1, H), W2)
    s2a, s2b = _msg_kernel()(y2a, y2b, row16, col16)
    part = _pool_kernel()(s2a, s2b, y2a, y2b, degp[0], degp[1], b2, batch_p)
    return _out_mm(part, Wc, bc)


# speedup vs baseline: 1.2496x; 1.2496x over previous
import functools

import jax
import jax.numpy as jnp
from jax import lax
from jax.experimental import pallas as pl
from jax.experimental.pallas import tpu as pltpu
from jax.experimental.pallas import tpu_sc as plsc

N = 10000
E = 320000
D = 128
H = 256
O = 16
G = 128

P = 10240
RPT = P // 32
RPS = P // 16
ECH = 80
MSG_NCH = E // 16 // ECH
DEG_NCH = E // 32 // ECH
BR = 512
HH = H // 2

_MESH = plsc.VectorSubcoreMesh(core_axis_name="c", subcore_axis_name="s")


def _zero_vmem(ref, rows, groups):
    z = jnp.zeros((16,), jnp.float32)

    def body(r, _):
        for g in range(groups):
            ref[r, pl.ds(g * 16, 16)] = z
        return 0

    lax.fori_loop(0, rows, body, 0)


def _deg_body(col32, degp, col_v, ones_v, z1, acc1):
    c = lax.axis_index("c")
    s = lax.axis_index("s")
    tid = c * 16 + s
    one = jnp.ones((16,), jnp.float32)
    for g in range(ECH // 16):
        ones_v[pl.ds(g * 16, 16)] = one
    z = jnp.zeros((16,), jnp.float32)

    def zb(i, _):
        z1[pl.ds(i * 16, 16)] = z
        return 0

    lax.fori_loop(0, RPS // 16, zb, 0)
    pltpu.sync_copy(z1, acc1.at[pl.ds(s * RPS, RPS)])
    pltpu.sync_copy(col32.at[tid], col_v)
    plsc.subcore_barrier()

    def chunk(j, _):
        pltpu.sync_copy(ones_v, acc1.at[col_v.at[j]], add=True)
        return 0

    lax.fori_loop(0, DEG_NCH, chunk, 0)
    plsc.subcore_barrier()
    pltpu.sync_copy(acc1.at[pl.ds(s * RPS, RPS)], degp.at[c, pl.ds(s * RPS, RPS)])


_deg_kernel = functools.partial(
    pl.kernel,
    _deg_body,
    out_type=jax.ShapeDtypeStruct((2, P), jnp.float32),
    mesh=_MESH,
    compiler_params=pltpu.CompilerParams(
        use_tc_tiling_on_sc=False, needs_layout_passes=False),
    scratch_types=[
        pltpu.VMEM((DEG_NCH, ECH), jnp.int32),
        pltpu.VMEM((ECH,), jnp.float32),
        pltpu.VMEM((RPS,), jnp.float32),
        pltpu.VMEM_SHARED((P,), jnp.float32),
    ],
)


MCH = 128
EPT = 20480
EP = EPT * 16
TCH = EPT // MCH
SCH = 32
NSC = TCH // SCH


def _msg_body(ya, yb, row16, col16, sa, sb, acc, row_sv, col_sv, gbuf,
              semg0, semg1, sems0, sems1):
    c = lax.axis_index("c")
    s = lax.axis_index("s")

    def run(src, dst):
        z = jnp.zeros((16,), jnp.float32)

        def zb(r, _):
            for g in range(HH // 16):
                gbuf[0, r, pl.ds(g * 16, 16)] = z
            return 0

        lax.fori_loop(0, MCH, zb, 0)
        for k in range(RPS // MCH):
            pltpu.async_copy(
                gbuf.at[0], acc.at[pl.ds(s * RPS + k * MCH, MCH)], sems1)
        for k in range(RPS // MCH):
            pltpu.make_async_copy(
                gbuf.at[0], acc.at[pl.ds(s * RPS + k * MCH, MCH)], sems1).wait()
        plsc.subcore_barrier()

        def superchunk(i, _):
            pltpu.sync_copy(row16.at[s, pl.ds(i * SCH, SCH)], row_sv)
            pltpu.sync_copy(col16.at[s, pl.ds(i * SCH, SCH)], col_sv)
            pltpu.async_copy(src.at[row_sv.at[0]], gbuf.at[0], semg0)
            pltpu.async_copy(src.at[row_sv.at[1]], gbuf.at[1], semg1)

            def pair(k, _):
                j0 = 2 * k
                j1 = j0 + 1
                pltpu.make_async_copy(
                    src.at[row_sv.at[j0]], gbuf.at[0], semg0).wait()
                pltpu.sync_copy(gbuf.at[0], acc.at[col_sv.at[j0]], add=True)

                @pl.when(k < SCH // 2 - 1)
                def _():
                    pltpu.async_copy(
                        src.at[row_sv.at[j0 + 2]], gbuf.at[0], semg0)

                pltpu.make_async_copy(
                    src.at[row_sv.at[j1]], gbuf.at[1], semg1).wait()
                pltpu.sync_copy(gbuf.at[1], acc.at[col_sv.at[j1]], add=True)

                @pl.when(k < SCH // 2 - 1)
                def _():
                    pltpu.async_copy(
                        src.at[row_sv.at[j1 + 2]], gbuf.at[1], semg1)

                return 0

            lax.fori_loop(0, SCH // 2, pair, 0)
            return 0

        lax.fori_loop(0, NSC, superchunk, 0)
        plsc.subcore_barrier()
        pltpu.sync_copy(acc.at[pl.ds(s * RPS, RPS)], dst.at[pl.ds(s * RPS, RPS)])

    @pl.when(c == 0)
    def _():
        run(ya, sa)

    @pl.when(c == 1)
    def _():
        run(yb, sb)


_msg_kernel = functools.partial(
    pl.kernel,
    _msg_body,
    out_type=(
        jax.ShapeDtypeStruct((P, HH), jnp.float32),
        jax.ShapeDtypeStruct((P, HH), jnp.float32),
    ),
    mesh=_MESH,
    compiler_params=pltpu.CompilerParams(
        use_tc_tiling_on_sc=False, needs_layout_passes=False),
    scratch_types=[
        pltpu.VMEM_SHARED((P, HH), jnp.float32),
        pltpu.VMEM((SCH, MCH), jnp.int32),
        pltpu.VMEM((SCH, MCH), jnp.int32),
        pltpu.VMEM((2, MCH, HH), jnp.float32),
        pltpu.SemaphoreType.DMA,
        pltpu.SemaphoreType.DMA,
        pltpu.SemaphoreType.DMA,
        pltpu.SemaphoreType.DMA,
    ],
)


def _rsqrt_newton(x):
    i = plsc.bitcast(x, jnp.int32)
    y = plsc.bitcast(jnp.int32(0x5F3759DF) - (i >> 1), jnp.float32)
    for _ in range(3):
        y = y * (1.5 - 0.5 * x * y * y)
    return y


def _pool_body(sa, sb, ya, yb, deg0, deg1, b2, batch, part_out,
               sbufa, sbufb, ybufa, ybufb, dv0, dv1, batch_v, b2_v, part):
    c = lax.axis_index("c")
    s = lax.axis_index("s")
    tid = c * 16 + s
    base = tid * RPT

    pltpu.sync_copy(deg0.at[pl.ds(base, RPT)], dv0.at[pl.ds(0, RPT)])
    pltpu.sync_copy(deg1.at[pl.ds(base, RPT)], dv1.at[pl.ds(0, RPT)])
    pltpu.sync_copy(batch.at[pl.ds(base, RPT)], batch_v.at[pl.ds(0, RPT)])
    pltpu.sync_copy(b2, b2_v)

    def dinvb(i, _):
        deg = dv0[pl.ds(i * 16, 16)] + dv1[pl.ds(i * 16, 16)] + 1.0
        dv0[pl.ds(i * 16, 16)] = _rsqrt_newton(deg)
        return 0

    lax.fori_loop(0, RPT // 16, dinvb, 0)
    _zero_vmem(part, G + 1, H // 16)
    b2vecs = [b2_v[pl.ds(g * 16, 16)] for g in range(H // 16)]

    for ch in range(RPT // 64):
        r0 = ch * 64
        pltpu.sync_copy(sa.at[pl.ds(base + r0, 64)], sbufa)
        pltpu.sync_copy(sb.at[pl.ds(base + r0, 64)], sbufb)
        pltpu.sync_copy(ya.at[pl.ds(base + r0, 64)], ybufa)
        pltpu.sync_copy(yb.at[pl.ds(base + r0, 64)], ybufb)

        def rowb(r, _):
            b_r = batch_v[pl.ds(r0 + r, 16)][0]
            dv = dv0[pl.ds(r0 + r, 16)][0]
            for g in range(H // 16):
                sl = pl.ds((g % 8) * 16, 16)
                if g < 8:
                    v = sbufa[r, sl] + ybufa[r, sl]
                else:
                    v = sbufb[r, sl] + ybufb[r, sl]
                hv = jnp.maximum(v * dv + b2vecs[g], 0.0)
                osl = pl.ds(g * 16, 16)
                part[b_r, osl] = jnp.maximum(part[b_r, osl], hv)
            return 0

        lax.fori_loop(0, 64, rowb, 0)

    pltpu.sync_copy(part, part_out.at[tid])


_pool_kernel = functools.partial(
    pl.kernel,
    _pool_body,
    out_type=jax.ShapeDtypeStruct((32, G + 1, H), jnp.float32),
    mesh=_MESH,
    compiler_params=pltpu.CompilerParams(
        use_tc_tiling_on_sc=False, needs_layout_passes=False),
    scratch_types=[
        pltpu.VMEM((64, HH), jnp.float32),
        pltpu.VMEM((64, HH), jnp.float32),
        pltpu.VMEM((64, HH), jnp.float32),
        pltpu.VMEM((64, HH), jnp.float32),
        pltpu.VMEM((RPT + 16,), jnp.float32),
        pltpu.VMEM((RPT + 16,), jnp.float32),
        pltpu.VMEM((RPT + 16,), jnp.int32),
        pltpu.VMEM((H,), jnp.float32),
        pltpu.VMEM((G + 1, H), jnp.float32),
    ],
)


def _mm1_body(x_ref, w_ref, degp_ref, ya_ref, yb_ref):
    z = jnp.dot(x_ref[...], w_ref[...], preferred_element_type=jnp.float32)
    deg = degp_ref[0, :] + degp_ref[1, :] + 1.0
    dinv = lax.rsqrt(deg)
    y = z * dinv[:, None]
    ya_ref[...] = y[:, :HH]
    yb_ref[...] = y[:, HH:]


def _mm1(x_p, w1, degp):
    return pl.pallas_call(
        _mm1_body,
        grid=(P // BR,),
        in_specs=[
            pl.BlockSpec((BR, D), lambda i: (i, 0)),
            pl.BlockSpec((D, H), lambda i: (0, 0)),
            pl.BlockSpec((2, BR), lambda i: (0, i)),
        ],
        out_specs=[
            pl.BlockSpec((BR, HH), lambda i: (i, 0)),
            pl.BlockSpec((BR, HH), lambda i: (i, 0)),
        ],
        out_shape=[
            jax.ShapeDtypeStruct((P, HH), jnp.float32),
            jax.ShapeDtypeStruct((P, HH), jnp.float32),
        ],
    )(x_p, w1, degp)


def _mm2_body(sa_ref, sb_ref, ya_ref, yb_ref, degp_ref, b1_ref, w2_ref,
              oa_ref, ob_ref):
    deg = degp_ref[0, :] + degp_ref[1, :] + 1.0
    dinv = lax.rsqrt(deg)[:, None]
    ha = jnp.maximum((sa_ref[...] + ya_ref[...]) * dinv + b1_ref[0, :HH], 0.0)
    hb = jnp.maximum((sb_ref[...] + yb_ref[...]) * dinv + b1_ref[0, HH:], 0.0)
    z = jnp.dot(ha, w2_ref[:HH, :], preferred_element_type=jnp.float32)
    z = z + jnp.dot(hb, w2_ref[HH:, :], preferred_element_type=jnp.float32)
    y = z * dinv
    oa_ref[...] = y[:, :HH]
    ob_ref[...] = y[:, HH:]


def _mm2(s1a, s1b, y1a, y1b, degp, b1, w2):
    return pl.pallas_call(
        _mm2_body,
        grid=(P // BR,),
        in_specs=[
            pl.BlockSpec((BR, HH), lambda i: (i, 0)),
            pl.BlockSpec((BR, HH), lambda i: (i, 0)),
            pl.BlockSpec((BR, HH), lambda i: (i, 0)),
            pl.BlockSpec((BR, HH), lambda i: (i, 0)),
            pl.BlockSpec((2, BR), lambda i: (0, i)),
            pl.BlockSpec((1, H), lambda i: (0, 0)),
            pl.BlockSpec((H, H), lambda i: (0, 0)),
        ],
        out_specs=[
            pl.BlockSpec((BR, HH), lambda i: (i, 0)),
            pl.BlockSpec((BR, HH), lambda i: (i, 0)),
        ],
        out_shape=[
            jax.ShapeDtypeStruct((P, HH), jnp.float32),
            jax.ShapeDtypeStruct((P, HH), jnp.float32),
        ],
    )(s1a, s1b, y1a, y1b, degp, b1, w2)


def _out_body(part_ref, wc_ref, bc_ref, out_ref):
    acc = part_ref[0, :G, :]
    for t in range(1, 32):
        acc = jnp.maximum(acc, part_ref[t, :G, :])
    out_ref[...] = (
        jnp.dot(acc, wc_ref[...], preferred_element_type=jnp.float32)
        + bc_ref[0, :]
    )


def _out_mm(part, wc, bc):
    return pl.pallas_call(
        _out_body,
        out_shape=jax.ShapeDtypeStruct((G, O), jnp.float32),
    )(part, wc, bc.reshape(1, O))


def kernel(feature_matrix, edge_index, batch, W1, b1, W2, b2, Wc, bc):
    row = edge_index[0]
    col = edge_index[1]
    x_p = jnp.pad(feature_matrix, ((0, P - N), (0, 0)))
    batch_p = jnp.concatenate(
        [batch, jnp.full((P - N,), G, dtype=batch.dtype)])
    npad_t = EPT - E // 16
    pad_idx = N + (jnp.arange(16 * npad_t, dtype=row.dtype) % (P - N))
    row16 = jnp.concatenate(
        [row.reshape(16, E // 16), pad_idx.reshape(16, npad_t)], axis=1
    ).reshape(16, TCH, MCH)
    col16 = jnp.concatenate(
        [col.reshape(16, E // 16), pad_idx.reshape(16, npad_t)], axis=1
    ).reshape(16, TCH, MCH)
    col32 = col.reshape(32, DEG_NCH, ECH)

    degp = _deg_kernel()(col32)
    y1a, y1b = _mm1(x_p, W1, degp)
    s1a, s1b = _msg_kernel()(y1a, y1b, row16, col16)
    y2a, y2b = _mm2(s1a, s1b, y1a, y1b, degp, b1.reshape(1, H), W2)
    s2a, s2b = _msg_kernel()(y2a, y2b, row16, col16)
    part = _pool_kernel()(s2a, s2b, y2a, y2b, degp[0], degp[1], b2, batch_p)
    return _out_mm(part, Wc, bc)


# speedup vs baseline: 1.2924x; 1.0343x over previous
import functools

import jax
import jax.numpy as jnp
from jax import lax
from jax.experimental import pallas as pl
from jax.experimental.pallas import tpu as pltpu
from jax.experimental.pallas import tpu_sc as plsc

N = 10000
E = 320000
D = 128
H = 256
O = 16
G = 128

P = 10240
RPT = P // 32
RPS = P // 16
ECH = 80
MSG_NCH = E // 16 // ECH
DEG_NCH = E // 32 // ECH
BR = 512
HH = H // 2

_MESH = plsc.VectorSubcoreMesh(core_axis_name="c", subcore_axis_name="s")


def _zero_vmem(ref, rows, groups):
    z = jnp.zeros((16,), jnp.float32)

    def body(r, _):
        for g in range(groups):
            ref[r, pl.ds(g * 16, 16)] = z
        return 0

    lax.fori_loop(0, rows, body, 0)


def _deg_body(col32, degp, col_v, ones_v, z1, acc1):
    c = lax.axis_index("c")
    s = lax.axis_index("s")
    tid = c * 16 + s
    one = jnp.ones((16,), jnp.float32)
    for g in range(ECH // 16):
        ones_v[pl.ds(g * 16, 16)] = one
    z = jnp.zeros((16,), jnp.float32)

    def zb(i, _):
        z1[pl.ds(i * 16, 16)] = z
        return 0

    lax.fori_loop(0, RPS // 16, zb, 0)
    pltpu.sync_copy(z1, acc1.at[pl.ds(s * RPS, RPS)])
    pltpu.sync_copy(col32.at[tid], col_v)
    plsc.subcore_barrier()

    def chunk(j, _):
        pltpu.sync_copy(ones_v, acc1.at[col_v.at[j]], add=True)
        return 0

    lax.fori_loop(0, DEG_NCH, chunk, 0)
    plsc.subcore_barrier()
    pltpu.sync_copy(acc1.at[pl.ds(s * RPS, RPS)], degp.at[c, pl.ds(s * RPS, RPS)])


_deg_kernel = functools.partial(
    pl.kernel,
    _deg_body,
    out_type=jax.ShapeDtypeStruct((2, P), jnp.float32),
    mesh=_MESH,
    compiler_params=pltpu.CompilerParams(
        use_tc_tiling_on_sc=False, needs_layout_passes=False),
    scratch_types=[
        pltpu.VMEM((DEG_NCH, ECH), jnp.int32),
        pltpu.VMEM((ECH,), jnp.float32),
        pltpu.VMEM((RPS,), jnp.float32),
        pltpu.VMEM_SHARED((P,), jnp.float32),
    ],
)


MCH = 128
EPT = 20480
EP = EPT * 16
TCH = EPT // MCH
SCH = 32
NSC = TCH // SCH


def _msg_body(ya, yb, row16, col16, sa, sb, acc, row_sv, col_sv, gbuf,
              semg0, semg1, sems0, sems1):
    c = lax.axis_index("c")
    s = lax.axis_index("s")

    def run(src, dst):
        z = jnp.zeros((16,), jnp.float32)

        def zb(r, _):
            for g in range(HH // 16):
                gbuf[0, r, pl.ds(g * 16, 16)] = z
            return 0

        lax.fori_loop(0, MCH, zb, 0)
        for k in range(RPS // MCH):
            pltpu.async_copy(
                gbuf.at[0], acc.at[pl.ds(s * RPS + k * MCH, MCH)], sems1)
        for k in range(RPS // MCH):
            pltpu.make_async_copy(
                gbuf.at[0], acc.at[pl.ds(s * RPS + k * MCH, MCH)], sems1).wait()
        plsc.subcore_barrier()

        def superchunk(i, _):
            pltpu.sync_copy(row16.at[s, pl.ds(i * SCH, SCH)], row_sv)
            pltpu.sync_copy(col16.at[s, pl.ds(i * SCH, SCH)], col_sv)
            pltpu.async_copy(src.at[row_sv.at[0]], gbuf.at[0], semg0)
            pltpu.async_copy(src.at[row_sv.at[1]], gbuf.at[1], semg1)

            def pair(k, _):
                j0 = 2 * k
                j1 = j0 + 1
                pltpu.make_async_copy(
                    src.at[row_sv.at[j0]], gbuf.at[0], semg0).wait()
                pltpu.sync_copy(gbuf.at[0], acc.at[col_sv.at[j0]], add=True)

                @pl.when(k < SCH // 2 - 1)
                def _():
                    pltpu.async_copy(
                        src.at[row_sv.at[j0 + 2]], gbuf.at[0], semg0)

                pltpu.make_async_copy(
                    src.at[row_sv.at[j1]], gbuf.at[1], semg1).wait()
                pltpu.sync_copy(gbuf.at[1], acc.at[col_sv.at[j1]], add=True)

                @pl.when(k < SCH // 2 - 1)
                def _():
                    pltpu.async_copy(
                        src.at[row_sv.at[j1 + 2]], gbuf.at[1], semg1)

                return 0

            lax.fori_loop(0, SCH // 2, pair, 0)
            return 0

        lax.fori_loop(0, NSC, superchunk, 0)
        plsc.subcore_barrier()
        pltpu.sync_copy(acc.at[pl.ds(s * RPS, RPS)], dst.at[pl.ds(s * RPS, RPS)])

    @pl.when(c == 0)
    def _():
        run(ya, sa)

    @pl.when(c == 1)
    def _():
        run(yb, sb)


_msg_kernel = functools.partial(
    pl.kernel,
    _msg_body,
    out_type=(
        jax.ShapeDtypeStruct((P, HH), jnp.float32),
        jax.ShapeDtypeStruct((P, HH), jnp.float32),
    ),
    mesh=_MESH,
    compiler_params=pltpu.CompilerParams(
        use_tc_tiling_on_sc=False, needs_layout_passes=False),
    scratch_types=[
        pltpu.VMEM_SHARED((P, HH), jnp.float32),
        pltpu.VMEM((SCH, MCH), jnp.int32),
        pltpu.VMEM((SCH, MCH), jnp.int32),
        pltpu.VMEM((2, MCH, HH), jnp.float32),
        pltpu.SemaphoreType.DMA,
        pltpu.SemaphoreType.DMA,
        pltpu.SemaphoreType.DMA,
        pltpu.SemaphoreType.DMA,
    ],
)


def _rsqrt_newton(x):
    i = plsc.bitcast(x, jnp.int32)
    y = plsc.bitcast(jnp.int32(0x5F3759DF) - (i >> 1), jnp.float32)
    for _ in range(3):
        y = y * (1.5 - 0.5 * x * y * y)
    return y


def _pool_body(sa, sb, ya, yb, deg0, deg1, b2, batch, part_out,
               sbufa, sbufb, ybufa, ybufb, dv0, dv1, batch_v, b2_v, part,
               semp0, semp1):
    c = lax.axis_index("c")
    s = lax.axis_index("s")
    tid = c * 16 + s
    base = tid * RPT
    NCHP = RPT // 64
    sems = (semp0, semp1)

    def stage(ch, slot):
        r0 = ch * 64
        pltpu.async_copy(sa.at[pl.ds(base + r0, 64)], sbufa.at[slot], sems[slot])
        pltpu.async_copy(sb.at[pl.ds(base + r0, 64)], sbufb.at[slot], sems[slot])
        pltpu.async_copy(ya.at[pl.ds(base + r0, 64)], ybufa.at[slot], sems[slot])
        pltpu.async_copy(yb.at[pl.ds(base + r0, 64)], ybufb.at[slot], sems[slot])

    def stage_wait(ch, slot):
        r0 = ch * 64
        pltpu.make_async_copy(
            sa.at[pl.ds(base + r0, 64)], sbufa.at[slot], sems[slot]).wait()
        pltpu.make_async_copy(
            sb.at[pl.ds(base + r0, 64)], sbufb.at[slot], sems[slot]).wait()
        pltpu.make_async_copy(
            ya.at[pl.ds(base + r0, 64)], ybufa.at[slot], sems[slot]).wait()
        pltpu.make_async_copy(
            yb.at[pl.ds(base + r0, 64)], ybufb.at[slot], sems[slot]).wait()

    stage(0, 0)
    pltpu.sync_copy(deg0.at[pl.ds(base, RPT)], dv0.at[pl.ds(0, RPT)])
    pltpu.sync_copy(deg1.at[pl.ds(base, RPT)], dv1.at[pl.ds(0, RPT)])
    pltpu.sync_copy(batch.at[pl.ds(base, RPT)], batch_v.at[pl.ds(0, RPT)])
    pltpu.sync_copy(b2, b2_v)

    def dinvb(i, _):
        deg = dv0[pl.ds(i * 16, 16)] + dv1[pl.ds(i * 16, 16)] + 1.0
        dv0[pl.ds(i * 16, 16)] = _rsqrt_newton(deg)
        return 0

    lax.fori_loop(0, RPT // 16, dinvb, 0)
    _zero_vmem(part, G + 1, H // 16)
    b2vecs = [b2_v[pl.ds(g * 16, 16)] for g in range(H // 16)]

    for ch in range(NCHP):
        slot = ch % 2
        if ch + 1 < NCHP:
            stage(ch + 1, 1 - slot)
        stage_wait(ch, slot)
        r0 = ch * 64

        def rowb(r, _):
            b_r = batch_v[pl.ds(r0 + r, 16)][0]
            dv = dv0[pl.ds(r0 + r, 16)][0]
            for g in range(H // 16):
                sl = pl.ds((g % 8) * 16, 16)
                if g < 8:
                    v = sbufa[slot, r, sl] + ybufa[slot, r, sl]
                else:
                    v = sbufb[slot, r, sl] + ybufb[slot, r, sl]
                hv = jnp.maximum(v * dv + b2vecs[g], 0.0)
                osl = pl.ds(g * 16, 16)
                part[b_r, osl] = jnp.maximum(part[b_r, osl], hv)
            return 0

        lax.fori_loop(0, 64, rowb, 0)

    pltpu.sync_copy(part, part_out.at[tid])


_pool_kernel = functools.partial(
    pl.kernel,
    _pool_body,
    out_type=jax.ShapeDtypeStruct((32, G + 1, H), jnp.float32),
    mesh=_MESH,
    compiler_params=pltpu.CompilerParams(
        use_tc_tiling_on_sc=False, needs_layout_passes=False),
    scratch_types=[
        pltpu.VMEM((2, 64, HH), jnp.float32),
        pltpu.VMEM((2, 64, HH), jnp.float32),
        pltpu.VMEM((2, 64, HH), jnp.float32),
        pltpu.VMEM((2, 64, HH), jnp.float32),
        pltpu.VMEM((RPT + 16,), jnp.float32),
        pltpu.VMEM((RPT + 16,), jnp.float32),
        pltpu.VMEM((RPT + 16,), jnp.int32),
        pltpu.VMEM((H,), jnp.float32),
        pltpu.VMEM((G + 1, H), jnp.float32),
        pltpu.SemaphoreType.DMA,
        pltpu.SemaphoreType.DMA,
    ],
)


def _mm1_body(x_ref, w_ref, degp_ref, ya_ref, yb_ref):
    z = jnp.dot(x_ref[...], w_ref[...], preferred_element_type=jnp.float32)
    deg = degp_ref[0, :] + degp_ref[1, :] + 1.0
    dinv = lax.rsqrt(deg)
    y = z * dinv[:, None]
    ya_ref[...] = y[:, :HH]
    yb_ref[...] = y[:, HH:]


def _mm1(x_p, w1, degp):
    return pl.pallas_call(
        _mm1_body,
        grid=(P // BR,),
        in_specs=[
            pl.BlockSpec((BR, D), lambda i: (i, 0)),
            pl.BlockSpec((D, H), lambda i: (0, 0)),
            pl.BlockSpec((2, BR), lambda i: (0, i)),
        ],
        out_specs=[
            pl.BlockSpec((BR, HH), lambda i: (i, 0)),
            pl.BlockSpec((BR, HH), lambda i: (i, 0)),
        ],
        out_shape=[
            jax.ShapeDtypeStruct((P, HH), jnp.float32),
            jax.ShapeDtypeStruct((P, HH), jnp.float32),
        ],
    )(x_p, w1, degp)


def _mm2_body(sa_ref, sb_ref, ya_ref, yb_ref, degp_ref, b1_ref, w2_ref,
              oa_ref, ob_ref):
    deg = degp_ref[0, :] + degp_ref[1, :] + 1.0
    dinv = lax.rsqrt(deg)[:, None]
    ha = jnp.maximum((sa_ref[...] + ya_ref[...]) * dinv + b1_ref[0, :HH], 0.0)
    hb = jnp.maximum((sb_ref[...] + yb_ref[...]) * dinv + b1_ref[0, HH:], 0.0)
    z = jnp.dot(ha, w2_ref[:HH, :], preferred_element_type=jnp.float32)
    z = z + jnp.dot(hb, w2_ref[HH:, :], preferred_element_type=jnp.float32)
    y = z * dinv
    oa_ref[...] = y[:, :HH]
    ob_ref[...] = y[:, HH:]


def _mm2(s1a, s1b, y1a, y1b, degp, b1, w2):
    return pl.pallas_call(
        _mm2_body,
        grid=(P // BR,),
        in_specs=[
            pl.BlockSpec((BR, HH), lambda i: (i, 0)),
            pl.BlockSpec((BR, HH), lambda i: (i, 0)),
            pl.BlockSpec((BR, HH), lambda i: (i, 0)),
            pl.BlockSpec((BR, HH), lambda i: (i, 0)),
            pl.BlockSpec((2, BR), lambda i: (0, i)),
            pl.BlockSpec((1, H), lambda i: (0, 0)),
            pl.BlockSpec((H, H), lambda i: (0, 0)),
        ],
        out_specs=[
            pl.BlockSpec((BR, HH), lambda i: (i, 0)),
            pl.BlockSpec((BR, HH), lambda i: (i, 0)),
        ],
        out_shape=[
            jax.ShapeDtypeStruct((P, HH), jnp.float32),
            jax.ShapeDtypeStruct((P, HH), jnp.float32),
        ],
    )(s1a, s1b, y1a, y1b, degp, b1, w2)


def _out_body(part_ref, wc_ref, bc_ref, out_ref):
    acc = part_ref[0, :G, :]
    for t in range(1, 32):
        acc = jnp.maximum(acc, part_ref[t, :G, :])
    out_ref[...] = (
        jnp.dot(acc, wc_ref[...], preferred_element_type=jnp.float32)
        + bc_ref[0, :]
    )


def _out_mm(part, wc, bc):
    return pl.pallas_call(
        _out_body,
        out_shape=jax.ShapeDtypeStruct((G, O), jnp.float32),
    )(part, wc, bc.reshape(1, O))


def kernel(feature_matrix, edge_index, batch, W1, b1, W2, b2, Wc, bc):
    row = edge_index[0]
    col = edge_index[1]
    x_p = jnp.pad(feature_matrix, ((0, P - N), (0, 0)))
    batch_p = jnp.concatenate(
        [batch, jnp.full((P - N,), G, dtype=batch.dtype)])
    npad_t = EPT - E // 16
    pad_idx = N + (jnp.arange(16 * npad_t, dtype=row.dtype) % (P - N))
    row16 = jnp.concatenate(
        [row.reshape(16, E // 16), pad_idx.reshape(16, npad_t)], axis=1
    ).reshape(16, TCH, MCH)
    col16 = jnp.concatenate(
        [col.reshape(16, E // 16), pad_idx.reshape(16, npad_t)], axis=1
    ).reshape(16, TCH, MCH)
    col32 = col.reshape(32, DEG_NCH, ECH)

    degp = _deg_kernel()(col32)
    y1a, y1b = _mm1(x_p, W1, degp)
    s1a, s1b = _msg_kernel()(y1a, y1b, row16, col16)
    y2a, y2b = _mm2(s1a, s1b, y1a, y1b, degp, b1.reshape(1, H), W2)
    s2a, s2b = _msg_kernel()(y2a, y2b, row16, col16)
    part = _pool_kernel()(s2a, s2b, y2a, y2b, degp[0], degp[1], b2, batch_p)
    return _out_mm(part, Wc, bc)


# speedup vs baseline: 1.3168x; 1.0188x over previous
import functools

import jax
import jax.numpy as jnp
from jax import lax
from jax.experimental import pallas as pl
from jax.experimental.pallas import tpu as pltpu
from jax.experimental.pallas import tpu_sc as plsc

N = 10000
E = 320000
D = 128
H = 256
O = 16
G = 128

P = 10240
RPT = P // 32
RPS = P // 16
ECH = 80
MSG_NCH = E // 16 // ECH
DCH = 128
DEPT = 10240
DEG_NCH = DEPT // DCH
BR = 512
HH = H // 2

_MESH = plsc.VectorSubcoreMesh(core_axis_name="c", subcore_axis_name="s")


def _zero_vmem(ref, rows, groups):
    z = jnp.zeros((16,), jnp.float32)

    def body(r, _):
        for g in range(groups):
            ref[r, pl.ds(g * 16, 16)] = z
        return 0

    lax.fori_loop(0, rows, body, 0)


def _deg_body(col32, degp, col_v, ones_v, z1, acc1, semd):
    c = lax.axis_index("c")
    s = lax.axis_index("s")
    tid = c * 16 + s
    one = jnp.ones((16,), jnp.float32)
    for g in range(DCH // 16):
        ones_v[pl.ds(g * 16, 16)] = one
    z = jnp.zeros((16,), jnp.float32)

    def zb(i, _):
        z1[pl.ds(i * 16, 16)] = z
        return 0

    lax.fori_loop(0, RPS // 16, zb, 0)
    pltpu.sync_copy(z1, acc1.at[pl.ds(s * RPS, RPS)])
    pltpu.sync_copy(col32.at[tid], col_v)
    plsc.subcore_barrier()

    def grp(i, _):
        for b in range(8):
            pltpu.async_copy(ones_v, acc1.at[col_v.at[i * 8 + b]], semd,
                             add=True)
        for b in range(8):
            pltpu.make_async_copy(
                ones_v, acc1.at[col_v.at[i * 8 + b]], semd).wait()
        return 0

    lax.fori_loop(0, DEG_NCH // 8, grp, 0)
    plsc.subcore_barrier()
    pltpu.sync_copy(acc1.at[pl.ds(s * RPS, RPS)], degp.at[c, pl.ds(s * RPS, RPS)])


_deg_kernel = functools.partial(
    pl.kernel,
    _deg_body,
    out_type=jax.ShapeDtypeStruct((2, P), jnp.float32),
    mesh=_MESH,
    compiler_params=pltpu.CompilerParams(
        use_tc_tiling_on_sc=False, needs_layout_passes=False),
    scratch_types=[
        pltpu.VMEM((DEG_NCH, DCH), jnp.int32),
        pltpu.VMEM((DCH,), jnp.float32),
        pltpu.VMEM((RPS,), jnp.float32),
        pltpu.VMEM_SHARED((P,), jnp.float32),
        pltpu.SemaphoreType.DMA,
    ],
)


MCH = 128
EPT = 20480
EP = EPT * 16
TCH = EPT // MCH
SCH = 40
NSC = TCH // SCH


def _msg_body(ya, yb, row16, col16, sa, sb, acc, row_sv, col_sv, gbuf,
              semg0, semg1, sems0, sems1):
    c = lax.axis_index("c")
    s = lax.axis_index("s")

    def run(src, dst):
        z = jnp.zeros((16,), jnp.float32)

        def zb(r, _):
            for g in range(HH // 16):
                gbuf[0, r, pl.ds(g * 16, 16)] = z
            return 0

        lax.fori_loop(0, MCH, zb, 0)
        for k in range(RPS // MCH):
            pltpu.async_copy(
                gbuf.at[0], acc.at[pl.ds(s * RPS + k * MCH, MCH)], sems1)
        for k in range(RPS // MCH):
            pltpu.make_async_copy(
                gbuf.at[0], acc.at[pl.ds(s * RPS + k * MCH, MCH)], sems1).wait()
        plsc.subcore_barrier()

        def superchunk(i, _):
            pltpu.sync_copy(row16.at[s, pl.ds(i * SCH, SCH)], row_sv)
            pltpu.sync_copy(col16.at[s, pl.ds(i * SCH, SCH)], col_sv)
            pltpu.async_copy(src.at[row_sv.at[0]], gbuf.at[0], semg0)
            pltpu.async_copy(src.at[row_sv.at[1]], gbuf.at[1], semg1)

            def pair(k, _):
                j0 = 2 * k
                j1 = j0 + 1
                pltpu.make_async_copy(
                    src.at[row_sv.at[j0]], gbuf.at[0], semg0).wait()
                pltpu.sync_copy(gbuf.at[0], acc.at[col_sv.at[j0]], add=True)

                @pl.when(k < SCH // 2 - 1)
                def _():
                    pltpu.async_copy(
                        src.at[row_sv.at[j0 + 2]], gbuf.at[0], semg0)

                pltpu.make_async_copy(
                    src.at[row_sv.at[j1]], gbuf.at[1], semg1).wait()
                pltpu.sync_copy(gbuf.at[1], acc.at[col_sv.at[j1]], add=True)

                @pl.when(k < SCH // 2 - 1)
                def _():
                    pltpu.async_copy(
                        src.at[row_sv.at[j1 + 2]], gbuf.at[1], semg1)

                return 0

            lax.fori_loop(0, SCH // 2, pair, 0)
            return 0

        lax.fori_loop(0, NSC, superchunk, 0)
        plsc.subcore_barrier()
        pltpu.sync_copy(acc.at[pl.ds(s * RPS, RPS)], dst.at[pl.ds(s * RPS, RPS)])

    @pl.when(c == 0)
    def _():
        run(ya, sa)

    @pl.when(c == 1)
    def _():
        run(yb, sb)


_msg_kernel = functools.partial(
    pl.kernel,
    _msg_body,
    out_type=(
        jax.ShapeDtypeStruct((P, HH), jnp.float32),
        jax.ShapeDtypeStruct((P, HH), jnp.float32),
    ),
    mesh=_MESH,
    compiler_params=pltpu.CompilerParams(
        use_tc_tiling_on_sc=False, needs_layout_passes=False),
    scratch_types=[
        pltpu.VMEM_SHARED((P, HH), jnp.float32),
        pltpu.VMEM((SCH, MCH), jnp.int32),
        pltpu.VMEM((SCH, MCH), jnp.int32),
        pltpu.VMEM((2, MCH, HH), jnp.float32),
        pltpu.SemaphoreType.DMA,
        pltpu.SemaphoreType.DMA,
        pltpu.SemaphoreType.DMA,
        pltpu.SemaphoreType.DMA,
    ],
)


def _rsqrt_newton(x):
    i = plsc.bitcast(x, jnp.int32)
    y = plsc.bitcast(jnp.int32(0x5F3759DF) - (i >> 1), jnp.float32)
    for _ in range(3):
        y = y * (1.5 - 0.5 * x * y * y)
    return y


def _pool_body(sa, sb, ya, yb, deg0, deg1, b2, batch, part_out,
               sbufa, sbufb, ybufa, ybufb, dv0, dv1, batch_v, b2_v, part,
               semp0, semp1):
    c = lax.axis_index("c")
    s = lax.axis_index("s")
    tid = c * 16 + s
    base = tid * RPT
    NCHP = RPT // 64
    sems = (semp0, semp1)

    def stage(ch, slot):
        r0 = ch * 64
        pltpu.async_copy(sa.at[pl.ds(base + r0, 64)], sbufa.at[slot], sems[slot])
        pltpu.async_copy(sb.at[pl.ds(base + r0, 64)], sbufb.at[slot], sems[slot])
        pltpu.async_copy(ya.at[pl.ds(base + r0, 64)], ybufa.at[slot], sems[slot])
        pltpu.async_copy(yb.at[pl.ds(base + r0, 64)], ybufb.at[slot], sems[slot])

    def stage_wait(ch, slot):
        r0 = ch * 64
        pltpu.make_async_copy(
            sa.at[pl.ds(base + r0, 64)], sbufa.at[slot], sems[slot]).wait()
        pltpu.make_async_copy(
            sb.at[pl.ds(base + r0, 64)], sbufb.at[slot], sems[slot]).wait()
        pltpu.make_async_copy(
            ya.at[pl.ds(base + r0, 64)], ybufa.at[slot], sems[slot]).wait()
        pltpu.make_async_copy(
            yb.at[pl.ds(base + r0, 64)], ybufb.at[slot], sems[slot]).wait()

    stage(0, 0)
    pltpu.sync_copy(deg0.at[pl.ds(base, RPT)], dv0.at[pl.ds(0, RPT)])
    pltpu.sync_copy(deg1.at[pl.ds(base, RPT)], dv1.at[pl.ds(0, RPT)])
    pltpu.sync_copy(batch.at[pl.ds(base, RPT)], batch_v.at[pl.ds(0, RPT)])
    pltpu.sync_copy(b2, b2_v)

    def dinvb(i, _):
        deg = dv0[pl.ds(i * 16, 16)] + dv1[pl.ds(i * 16, 16)] + 1.0
        dv0[pl.ds(i * 16, 16)] = _rsqrt_newton(deg)
        return 0

    lax.fori_loop(0, RPT // 16, dinvb, 0)
    _zero_vmem(part, G + 1, H // 16)
    b2vecs = [b2_v[pl.ds(g * 16, 16)] for g in range(H // 16)]

    for ch in range(NCHP):
        slot = ch % 2
        if ch + 1 < NCHP:
            stage(ch + 1, 1 - slot)
        stage_wait(ch, slot)
        r0 = ch * 64

        def rowb(r, _):
            b_r = batch_v[pl.ds(r0 + r, 16)][0]
            dv = dv0[pl.ds(r0 + r, 16)][0]
            for g in range(H // 16):
                sl = pl.ds((g % 8) * 16, 16)
                if g < 8:
                    v = sbufa[slot, r, sl] + ybufa[slot, r, sl]
                else:
                    v = sbufb[slot, r, sl] + ybufb[slot, r, sl]
                hv = jnp.maximum(v * dv + b2vecs[g], 0.0)
                osl = pl.ds(g * 16, 16)
                part[b_r, osl] = jnp.maximum(part[b_r, osl], hv)
            return 0

        lax.fori_loop(0, 64, rowb, 0)

    pltpu.sync_copy(part, part_out.at[tid])


_pool_kernel = functools.partial(
    pl.kernel,
    _pool_body,
    out_type=jax.ShapeDtypeStruct((32, G + 1, H), jnp.float32),
    mesh=_MESH,
    compiler_params=pltpu.CompilerParams(
        use_tc_tiling_on_sc=False, needs_layout_passes=False),
    scratch_types=[
        pltpu.VMEM((2, 64, HH), jnp.float32),
        pltpu.VMEM((2, 64, HH), jnp.float32),
        pltpu.VMEM((2, 64, HH), jnp.float32),
        pltpu.VMEM((2, 64, HH), jnp.float32),
        pltpu.VMEM((RPT + 16,), jnp.float32),
        pltpu.VMEM((RPT + 16,), jnp.float32),
        pltpu.VMEM((RPT + 16,), jnp.int32),
        pltpu.VMEM((H,), jnp.float32),
        pltpu.VMEM((G + 1, H), jnp.float32),
        pltpu.SemaphoreType.DMA,
        pltpu.SemaphoreType.DMA,
    ],
)


def _mm1_body(x_ref, w_ref, degp_ref, ya_ref, yb_ref):
    z = jnp.dot(x_ref[...], w_ref[...], preferred_element_type=jnp.float32)
    deg = degp_ref[0, :] + degp_ref[1, :] + 1.0
    dinv = lax.rsqrt(deg)
    y = z * dinv[:, None]
    ya_ref[...] = y[:, :HH]
    yb_ref[...] = y[:, HH:]


def _mm1(x_p, w1, degp):
    return pl.pallas_call(
        _mm1_body,
        grid=(P // BR,),
        in_specs=[
            pl.BlockSpec((BR, D), lambda i: (i, 0)),
            pl.BlockSpec((D, H), lambda i: (0, 0)),
            pl.BlockSpec((2, BR), lambda i: (0, i)),
        ],
        out_specs=[
            pl.BlockSpec((BR, HH), lambda i: (i, 0)),
            pl.BlockSpec((BR, HH), lambda i: (i, 0)),
        ],
        out_shape=[
            jax.ShapeDtypeStruct((P, HH), jnp.float32),
            jax.ShapeDtypeStruct((P, HH), jnp.float32),
        ],
    )(x_p, w1, degp)


def _mm2_body(sa_ref, sb_ref, ya_ref, yb_ref, degp_ref, b1_ref, w2_ref,
              oa_ref, ob_ref):
    deg = degp_ref[0, :] + degp_ref[1, :] + 1.0
    dinv = lax.rsqrt(deg)[:, None]
    ha = jnp.maximum((sa_ref[...] + ya_ref[...]) * dinv + b1_ref[0, :HH], 0.0)
    hb = jnp.maximum((sb_ref[...] + yb_ref[...]) * dinv + b1_ref[0, HH:], 0.0)
    z = jnp.dot(ha, w2_ref[:HH, :], preferred_element_type=jnp.float32)
    z = z + jnp.dot(hb, w2_ref[HH:, :], preferred_element_type=jnp.float32)
    y = z * dinv
    oa_ref[...] = y[:, :HH]
    ob_ref[...] = y[:, HH:]


def _mm2(s1a, s1b, y1a, y1b, degp, b1, w2):
    return pl.pallas_call(
        _mm2_body,
        grid=(P // BR,),
        in_specs=[
            pl.BlockSpec((BR, HH), lambda i: (i, 0)),
            pl.BlockSpec((BR, HH), lambda i: (i, 0)),
            pl.BlockSpec((BR, HH), lambda i: (i, 0)),
            pl.BlockSpec((BR, HH), lambda i: (i, 0)),
            pl.BlockSpec((2, BR), lambda i: (0, i)),
            pl.BlockSpec((1, H), lambda i: (0, 0)),
            pl.BlockSpec((H, H), lambda i: (0, 0)),
        ],
        out_specs=[
            pl.BlockSpec((BR, HH), lambda i: (i, 0)),
            pl.BlockSpec((BR, HH), lambda i: (i, 0)),
        ],
        out_shape=[
            jax.ShapeDtypeStruct((P, HH), jnp.float32),
            jax.ShapeDtypeStruct((P, HH), jnp.float32),
        ],
    )(s1a, s1b, y1a, y1b, degp, b1, w2)


def _out_body(part_ref, wc_ref, bc_ref, out_ref):
    acc = part_ref[0, :G, :]
    for t in range(1, 32):
        acc = jnp.maximum(acc, part_ref[t, :G, :])
    out_ref[...] = (
        jnp.dot(acc, wc_ref[...], preferred_element_type=jnp.float32)
        + bc_ref[0, :]
    )


def _out_mm(part, wc, bc):
    return pl.pallas_call(
        _out_body,
        out_shape=jax.ShapeDtypeStruct((G, O), jnp.float32),
    )(part, wc, bc.reshape(1, O))


def kernel(feature_matrix, edge_index, batch, W1, b1, W2, b2, Wc, bc):
    row = edge_index[0]
    col = edge_index[1]
    x_p = jnp.pad(feature_matrix, ((0, P - N), (0, 0)))
    batch_p = jnp.concatenate(
        [batch, jnp.full((P - N,), G, dtype=batch.dtype)])
    npad_t = EPT - E // 16
    pad_idx = N + (jnp.arange(16 * npad_t, dtype=row.dtype) % (P - N))
    row16 = jnp.concatenate(
        [row.reshape(16, E // 16), pad_idx.reshape(16, npad_t)], axis=1
    ).reshape(16, TCH, MCH)
    col16 = jnp.concatenate(
        [col.reshape(16, E // 16), pad_idx.reshape(16, npad_t)], axis=1
    ).reshape(16, TCH, MCH)
    dpad_t = DEPT - E // 32
    dpad = N + (jnp.arange(32 * dpad_t, dtype=col.dtype) % (P - N))
    col32 = jnp.concatenate(
        [col.reshape(32, E // 32), dpad.reshape(32, dpad_t)], axis=1
    ).reshape(32, DEG_NCH, DCH)

    degp = _deg_kernel()(col32)
    y1a, y1b = _mm1(x_p, W1, degp)
    s1a, s1b = _msg_kernel()(y1a, y1b, row16, col16)
    y2a, y2b = _mm2(s1a, s1b, y1a, y1b, degp, b1.reshape(1, H), W2)
    s2a, s2b = _msg_kernel()(y2a, y2b, row16, col16)
    part = _pool_kernel()(s2a, s2b, y2a, y2b, degp[0], degp[1], b2, batch_p)
    return _out_mm(part, Wc, bc)
